# baseline (speedup 1.0000x reference)
import functools

import jax
import jax.numpy as jnp
from jax import lax
from jax.experimental import pallas as pl
from jax.experimental.pallas import tpu as pltpu
from jax.experimental.pallas import tpu_sc as plsc

V = 10000
E = 160000
LN = 128
S = 32
C = 40
MU = 0.9

NC = 2
NS = 16
NW = NC * NS
EPW = E // NW
CH = 8
NCHUNK = EPW // CH
VP = 10240
VPS = VP // NS
BV = 1000
LANES = 16
UNROLL = 4

def _hist_body(xn_hbm, out_hbm, idx_v, cnt_v):
    cid = lax.axis_index("c")
    sid = lax.axis_index("s")
    wid = sid * NC + cid
    pltpu.sync_copy(xn_hbm.at[wid], idx_v)

    def zero_body(i, carry):
        cnt_v[pl.ds(i * LANES, LANES)] = jnp.zeros((LANES,), jnp.float32)
        return carry

    lax.fori_loop(0, V // LANES, zero_body, 0)

    ones = jnp.ones((LANES,), jnp.float32)

    def hist_body(k, carry):
        v16 = idx_v[pl.ds(k * LANES, LANES)]
        plsc.addupdate_scatter(cnt_v, [v16], ones)
        return carry

    nfull = EPW // LANES
    lax.fori_loop(0, nfull, hist_body, 0)
    rem = EPW - nfull * LANES
    if rem:
        v16 = idx_v[pl.ds(EPW - LANES, LANES)]
        mask = lax.iota(jnp.int32, LANES) >= (LANES - rem)
        plsc.addupdate_scatter(cnt_v, [v16], ones, mask=mask)
    pltpu.sync_copy(cnt_v, out_hbm.at[wid])


def _mm_body(feat, w1, w2, bxi, wr, brou, cnt_t, p_out, q_out, h1_out):
    f = feat[...]
    p_out[...] = jnp.dot(f, w1[...], preferred_element_type=jnp.float32)
    q_out[...] = jnp.dot(f, w2[...], preferred_element_type=jnp.float32) + bxi[...]
    r = jnp.tanh(jnp.dot(f, wr[...], preferred_element_type=jnp.float32) + brou[...])
    cnt = jnp.sum(cnt_t[...], axis=1, keepdims=True)
    h1_out[...] = cnt * r


_mm_call = pl.pallas_call(
    _mm_body,
    grid=(V // BV,),
    in_specs=[
        pl.BlockSpec((BV, LN), lambda i: (i, 0)),
        pl.BlockSpec((LN, S * S), lambda i: (0, 0)),
        pl.BlockSpec((LN, S * S), lambda i: (0, 0)),
        pl.BlockSpec((1, S * S), lambda i: (0, 0)),
        pl.BlockSpec((LN, S), lambda i: (0, 0)),
        pl.BlockSpec((1, S), lambda i: (0, 0)),
        pl.BlockSpec((BV, NW), lambda i: (i, 0)),
    ],
    out_specs=[
        pl.BlockSpec((BV, S * S), lambda i: (i, 0)),
        pl.BlockSpec((BV, S * S), lambda i: (i, 0)),
        pl.BlockSpec((BV, S), lambda i: (i, 0)),
    ],
    out_shape=[
        jax.ShapeDtypeStruct((V, S * S), jnp.float32),
        jax.ShapeDtypeStruct((V, S * S), jnp.float32),
        jax.ShapeDtypeStruct((V, S), jnp.float32),
    ],
)


def _edge_body(pb_hbm, qb_hbm, h1_hbm, xn_hbm, xs_hbm, dg_hbm, out_hbm,
                 idxn, idxs, dgb, pbuf, qbuf, h1g, ubuf, zbuf, hsh,
                 sp0, sp1, sq0, sq1, sh0, sh1):
    cid = lax.axis_index("c")
    sid = lax.axis_index("s")
    wid = sid * NC + cid
    semp = (sp0, sp1)
    semq = (sq0, sq1)
    semh = (sh0, sh1)

    pltpu.sync_copy(xn_hbm.at[wid], idxn)
    pltpu.sync_copy(xs_hbm.at[wid], idxs)
    pltpu.sync_copy(dg_hbm.at[wid], dgb)

    zero = jnp.zeros((LANES,), jnp.float32)

    def zb(i, carry):
        for t in range(S // LANES):
            zbuf[i, pl.ds(t * LANES, LANES)] = zero
        return carry

    lax.fori_loop(0, VPS // 4, zb, 0)

    def zs(k, carry):
        off = pl.multiple_of(sid * VPS + k * (VPS // 4), 8)
        pltpu.sync_copy(zbuf, hsh.at[pl.ds(off, VPS // 4)])
        return carry

    lax.fori_loop(0, 4, zs, 0)
    plsc.subcore_barrier()

    def issue(cc, b):
        pltpu.async_copy(pb_hbm.at[idxn.at[cc]], pbuf.at[b], semp[b])
        pltpu.async_copy(qb_hbm.at[idxs.at[cc]], qbuf.at[b], semq[b])
        pltpu.async_copy(h1_hbm.at[idxn.at[cc]], h1g.at[b], semh[b])

    def compute(cc, b):
        pltpu.make_async_copy(pb_hbm.at[idxn.at[cc]], pbuf.at[b], semp[b]).wait()
        pltpu.make_async_copy(qb_hbm.at[idxs.at[cc]], qbuf.at[b], semq[b]).wait()
        pltpu.make_async_copy(h1_hbm.at[idxn.at[cc]], h1g.at[b], semh[b]).wait()

        def edge_body(r, carry):
            eidx = jnp.full((LANES,), cc * CH + r, jnp.int32)
            dvec = plsc.load_gather(dgb, [eidx])
            svec = (2.0 * MU / S) / dvec
            hs0 = h1g[b, r, pl.ds(0, LANES)] * svec
            hs1 = h1g[b, r, pl.ds(LANES, LANES)] * svec
            hsum = 0.5 * jnp.sum(hs0 + hs1, axis=0)
            acc0 = [jnp.full((LANES,), hsum, jnp.float32),
                    jnp.zeros((LANES,), jnp.float32),
                    jnp.zeros((LANES,), jnp.float32),
                    jnp.zeros((LANES,), jnp.float32)]
            acc1 = [jnp.full((LANES,), hsum, jnp.float32),
                    jnp.zeros((LANES,), jnp.float32),
                    jnp.zeros((LANES,), jnp.float32),
                    jnp.zeros((LANES,), jnp.float32)]
            for col in range(S):
                base = col * S
                p0 = pbuf[b, r, pl.ds(base, LANES)]
                p1 = pbuf[b, r, pl.ds(base + LANES, LANES)]
                q0 = qbuf[b, r, pl.ds(base, LANES)]
                q1 = qbuf[b, r, pl.ds(base + LANES, LANES)]
                e0 = jnp.exp(p0 + q0)
                e1 = jnp.exp(p1 + q1)
                f0 = e0 + 1.0
                f1 = e1 + 1.0
                rm = 1.0 / (f0 * f1)
                r0 = f1 * rm
                r1 = f0 * rm
                hv2 = (hs0 if col < LANES else hs1)[col % LANES]
                k = col % 4
                acc0[k] = acc0[k] - hv2 * r0
                acc1[k] = acc1[k] - hv2 * r1
            u0 = (acc0[0] + acc0[1]) + (acc0[2] + acc0[3])
            u1 = (acc1[0] + acc1[1]) + (acc1[2] + acc1[3])
            ubuf[r, pl.ds(0, LANES)] = u0
            ubuf[r, pl.ds(LANES, LANES)] = u1
            return carry

        lax.fori_loop(0, CH, edge_body, 0)
        pltpu.sync_copy(ubuf, hsh.at[idxn.at[cc]], add=True)

    issue(0, 0)

    def pair_body(k, carry):
        for b in range(2):
            cc = 2 * k + b
            issue(cc + 1, 1 - b)
            compute(cc, b)
        return carry

    lax.fori_loop(0, (NCHUNK - 1) // 2, pair_body, 0)
    compute(NCHUNK - 1, 0)

    plsc.subcore_barrier()
    doff = pl.multiple_of(sid * VPS, 8)
    pltpu.sync_copy(hsh.at[pl.ds(doff, VPS)],
                    out_hbm.at[cid, pl.ds(doff, VPS)])


def _out_body(hacc, h1, wo, bo, out):
    h2 = hacc[0, :, :S] + hacc[1, :, :S] + h1[:, :S]
    logits = jnp.dot(h2, wo[...], preferred_element_type=jnp.float32) + bo[...]
    m = jnp.max(logits, axis=-1, keepdims=True)
    z = logits - m
    lse = jnp.log(jnp.sum(jnp.exp(z), axis=-1, keepdims=True))
    out[...] = z - lse


_out_call = pl.pallas_call(
    _out_body,
    grid=(V // BV,),
    in_specs=[
        pl.BlockSpec((NC, BV, S), lambda i: (0, i, 0)),
        pl.BlockSpec((BV, S), lambda i: (i, 0)),
        pl.BlockSpec((S, C), lambda i: (0, 0)),
        pl.BlockSpec((1, C), lambda i: (0, 0)),
    ],
    out_specs=pl.BlockSpec((BV, C), lambda i: (i, 0)),
    out_shape=jax.ShapeDtypeStruct((V, C), jnp.float32),
)


@functools.lru_cache(maxsize=1)
def _sc_kernels():
    mesh = plsc.VectorSubcoreMesh(
        core_axis_name="c", subcore_axis_name="s",
        num_cores=NC, num_subcores=NS)
    params = pltpu.CompilerParams(needs_layout_passes=False)
    hist = pl.kernel(
        _hist_body,
        out_type=jax.ShapeDtypeStruct((NW, V), jnp.float32),
        mesh=mesh,
        compiler_params=params,
        scratch_types=[
            pltpu.VMEM((EPW,), jnp.int32),
            pltpu.VMEM((V,), jnp.float32),
        ],
    )
    edge = pl.kernel(
        _edge_body,
        out_type=jax.ShapeDtypeStruct((NC, VP, S), jnp.float32),
        mesh=mesh,
        compiler_params=pltpu.CompilerParams(
            needs_layout_passes=False, use_tc_tiling_on_sc=False),
        scratch_types=[
            pltpu.VMEM((NCHUNK, CH), jnp.int32),
            pltpu.VMEM((NCHUNK, CH), jnp.int32),
            pltpu.VMEM((EPW,), jnp.float32),
            pltpu.VMEM((2, CH, S * S), jnp.float32),
            pltpu.VMEM((2, CH, S * S), jnp.float32),
            pltpu.VMEM((2, CH, S), jnp.float32),
            pltpu.VMEM((CH, S), jnp.float32),
            pltpu.VMEM((VPS // 4, S), jnp.float32),
            pltpu.VMEM_SHARED((VP, S), jnp.float32),
            pltpu.SemaphoreType.DMA,
            pltpu.SemaphoreType.DMA,
            pltpu.SemaphoreType.DMA,
            pltpu.SemaphoreType.DMA,
            pltpu.SemaphoreType.DMA,
            pltpu.SemaphoreType.DMA,
        ],
    )
    return hist, edge


def kernel(feat_Matrix, X_Node, X_Neis, dg_list, W_xi, b_xi, W_rou, b_rou,
           W_out, b_out):
    w_t = W_xi.reshape(S, S, 2 * LN).transpose(1, 0, 2).reshape(S * S, 2 * LN)
    w1 = 2.0 * w_t[:, :LN].T
    w2 = 2.0 * w_t[:, LN:].T
    bxi = 2.0 * b_xi.reshape(S, S).T.reshape(1, S * S)
    wr = W_rou.T
    brou = b_rou.reshape(1, S)
    wo = W_out.T
    bo = b_out.reshape(1, C)

    xn = X_Node.astype(jnp.int32)
    xs = X_Neis.astype(jnp.int32)
    xn_w = xn.reshape(NW, EPW)
    xn_c = xn.reshape(NW, NCHUNK, CH)
    xs_c = xs.reshape(NW, NCHUNK, CH)
    dg_w = dg_list.astype(jnp.float32).reshape(NW, EPW)

    hist_kernel, edge_kernel = _sc_kernels()
    cnt32 = hist_kernel(xn_w)
    cnt_t = cnt32.T
    pb, qb, h1 = _mm_call(feat_Matrix, w1, w2, bxi, wr, brou, cnt_t)
    hacc = edge_kernel(pb, qb, h1, xn_c, xs_c, dg_w)
    return _out_call(hacc, h1, wo, bo)


# speedup vs baseline: 2.2480x; 2.2480x over previous
import functools

import jax
import jax.numpy as jnp
from jax import lax
from jax.experimental import pallas as pl
from jax.experimental.pallas import tpu as pltpu
from jax.experimental.pallas import tpu_sc as plsc
from jax._src.pallas.mosaic import lowering as _mosaic_lowering

_mosaic_lowering.lowering_rules[pltpu.CoreType.SC_VECTOR_SUBCORE][lax.tanh_p] = (
    _mosaic_lowering.lowering_rules[pltpu.CoreType.TC][lax.tanh_p])

V = 10000
E = 160000
LN = 128
S = 32
C = 40
MU = 0.9

NC = 2
NS = 16
NW = NC * NS
EPW = E // NW
CH = 8
NCHUNK = EPW // CH
VP = 10240
VPS = VP // NS
BV = 1000
LANES = 16
UNROLL = 4

def _hist_body(xn_hbm, out_hbm, idx_v, cnt_v):
    cid = lax.axis_index("c")
    sid = lax.axis_index("s")
    wid = sid * NC + cid
    pltpu.sync_copy(xn_hbm.at[wid], idx_v)

    def zero_body(i, carry):
        cnt_v[pl.ds(i * LANES, LANES)] = jnp.zeros((LANES,), jnp.float32)
        return carry

    lax.fori_loop(0, V // LANES, zero_body, 0)

    ones = jnp.ones((LANES,), jnp.float32)

    def hist_body(k, carry):
        v16 = idx_v[pl.ds(k * LANES, LANES)]
        plsc.addupdate_scatter(cnt_v, [v16], ones)
        return carry

    nfull = EPW // LANES
    lax.fori_loop(0, nfull, hist_body, 0)
    rem = EPW - nfull * LANES
    if rem:
        v16 = idx_v[pl.ds(EPW - LANES, LANES)]
        mask = lax.iota(jnp.int32, LANES) >= (LANES - rem)
        plsc.addupdate_scatter(cnt_v, [v16], ones, mask=mask)
    pltpu.sync_copy(cnt_v, out_hbm.at[wid])


def _mm_body(feat, w1, w2, bxi, wr, brou, cnt_t, p_out, q_out, h1_out):
    f = feat[...]
    p_out[...] = jnp.dot(f, w1[...], preferred_element_type=jnp.float32)
    q_out[...] = jnp.dot(f, w2[...], preferred_element_type=jnp.float32) + bxi[...]
    r = jnp.tanh(jnp.dot(f, wr[...], preferred_element_type=jnp.float32) + brou[...])
    cnt = jnp.sum(cnt_t[...], axis=1, keepdims=True)
    h1_out[...] = cnt * r


_mm_call = pl.pallas_call(
    _mm_body,
    grid=(V // BV,),
    in_specs=[
        pl.BlockSpec((BV, LN), lambda i: (i, 0)),
        pl.BlockSpec((LN, S * S), lambda i: (0, 0)),
        pl.BlockSpec((LN, S * S), lambda i: (0, 0)),
        pl.BlockSpec((1, S * S), lambda i: (0, 0)),
        pl.BlockSpec((LN, S), lambda i: (0, 0)),
        pl.BlockSpec((1, S), lambda i: (0, 0)),
        pl.BlockSpec((BV, NW), lambda i: (i, 0)),
    ],
    out_specs=[
        pl.BlockSpec((BV, S * S), lambda i: (i, 0)),
        pl.BlockSpec((BV, S * S), lambda i: (i, 0)),
        pl.BlockSpec((BV, S), lambda i: (i, 0)),
    ],
    out_shape=[
        jax.ShapeDtypeStruct((V, S * S), jnp.float32),
        jax.ShapeDtypeStruct((V, S * S), jnp.float32),
        jax.ShapeDtypeStruct((V, S), jnp.float32),
    ],
)


def _edge_body(pb_hbm, qb_hbm, h1_hbm, xn_hbm, xs_hbm, dg_hbm, out_hbm,
                 idxn, idxs, dgb, pbuf, qbuf, h1g, ubuf, zbuf, hsh,
                 sp0, sp1, sq0, sq1, sh0, sh1):
    cid = lax.axis_index("c")
    sid = lax.axis_index("s")
    wid = sid * NC + cid
    semp = (sp0, sp1)
    semq = (sq0, sq1)
    semh = (sh0, sh1)

    pltpu.sync_copy(xn_hbm.at[wid], idxn)
    pltpu.sync_copy(xs_hbm.at[wid], idxs)
    pltpu.sync_copy(dg_hbm.at[wid], dgb)

    zero = jnp.zeros((LANES,), jnp.float32)

    def zb(i, carry):
        for t in range(S // LANES):
            zbuf[i, pl.ds(t * LANES, LANES)] = zero
        return carry

    lax.fori_loop(0, VPS // 4, zb, 0)

    def zs(k, carry):
        off = pl.multiple_of(sid * VPS + k * (VPS // 4), 8)
        pltpu.sync_copy(zbuf, hsh.at[pl.ds(off, VPS // 4)])
        return carry

    lax.fori_loop(0, 4, zs, 0)
    plsc.subcore_barrier()

    def issue(cc, b):
        pltpu.async_copy(pb_hbm.at[idxn.at[cc]], pbuf.at[b], semp[b])
        pltpu.async_copy(qb_hbm.at[idxs.at[cc]], qbuf.at[b], semq[b])
        pltpu.async_copy(h1_hbm.at[idxn.at[cc]], h1g.at[b], semh[b])

    def compute(cc, b):
        pltpu.make_async_copy(pb_hbm.at[idxn.at[cc]], pbuf.at[b], semp[b]).wait()
        pltpu.make_async_copy(qb_hbm.at[idxs.at[cc]], qbuf.at[b], semq[b]).wait()
        pltpu.make_async_copy(h1_hbm.at[idxn.at[cc]], h1g.at[b], semh[b]).wait()

        def edge_body(r, carry):
            eidx = jnp.full((LANES,), cc * CH + r, jnp.int32)
            dvec = plsc.load_gather(dgb, [eidx])
            svec = (MU / S) / dvec
            hs0 = h1g[b, r, pl.ds(0, LANES)] * svec
            hs1 = h1g[b, r, pl.ds(LANES, LANES)] * svec
            acc0 = [jnp.zeros((LANES,), jnp.float32) for _ in range(4)]
            acc1 = [jnp.zeros((LANES,), jnp.float32) for _ in range(4)]
            for col in range(S):
                base = col * S
                p0 = pbuf[b, r, pl.ds(base, LANES)]
                p1 = pbuf[b, r, pl.ds(base + LANES, LANES)]
                q0 = qbuf[b, r, pl.ds(base, LANES)]
                q1 = qbuf[b, r, pl.ds(base + LANES, LANES)]
                t0 = jnp.tanh(p0 + q0)
                t1 = jnp.tanh(p1 + q1)
                hv = (hs0 if col < LANES else hs1)[col % LANES]
                k = col % 4
                acc0[k] = acc0[k] + t0 * hv
                acc1[k] = acc1[k] + t1 * hv
            u0 = (acc0[0] + acc0[1]) + (acc0[2] + acc0[3])
            u1 = (acc1[0] + acc1[1]) + (acc1[2] + acc1[3])
            ubuf[r, pl.ds(0, LANES)] = u0
            ubuf[r, pl.ds(LANES, LANES)] = u1
            return carry

        lax.fori_loop(0, CH, edge_body, 0)
        pltpu.sync_copy(ubuf, hsh.at[idxn.at[cc]], add=True)

    issue(0, 0)

    def pair_body(k, carry):
        for b in range(2):
            cc = 2 * k + b
            issue(cc + 1, 1 - b)
            compute(cc, b)
        return carry

    lax.fori_loop(0, (NCHUNK - 1) // 2, pair_body, 0)
    compute(NCHUNK - 1, 0)

    plsc.subcore_barrier()
    doff = pl.multiple_of(sid * VPS, 8)
    pltpu.sync_copy(hsh.at[pl.ds(doff, VPS)],
                    out_hbm.at[cid, pl.ds(doff, VPS)])


def _out_body(hacc, h1, wo, bo, out):
    h2 = hacc[0, :, :S] + hacc[1, :, :S] + h1[:, :S]
    logits = jnp.dot(h2, wo[...], preferred_element_type=jnp.float32) + bo[...]
    m = jnp.max(logits, axis=-1, keepdims=True)
    z = logits - m
    lse = jnp.log(jnp.sum(jnp.exp(z), axis=-1, keepdims=True))
    out[...] = z - lse


_out_call = pl.pallas_call(
    _out_body,
    grid=(V // BV,),
    in_specs=[
        pl.BlockSpec((NC, BV, S), lambda i: (0, i, 0)),
        pl.BlockSpec((BV, S), lambda i: (i, 0)),
        pl.BlockSpec((S, C), lambda i: (0, 0)),
        pl.BlockSpec((1, C), lambda i: (0, 0)),
    ],
    out_specs=pl.BlockSpec((BV, C), lambda i: (i, 0)),
    out_shape=jax.ShapeDtypeStruct((V, C), jnp.float32),
)


@functools.lru_cache(maxsize=1)
def _sc_kernels():
    mesh = plsc.VectorSubcoreMesh(
        core_axis_name="c", subcore_axis_name="s",
        num_cores=NC, num_subcores=NS)
    params = pltpu.CompilerParams(needs_layout_passes=False)
    hist = pl.kernel(
        _hist_body,
        out_type=jax.ShapeDtypeStruct((NW, V), jnp.float32),
        mesh=mesh,
        compiler_params=params,
        scratch_types=[
            pltpu.VMEM((EPW,), jnp.int32),
            pltpu.VMEM((V,), jnp.float32),
        ],
    )
    edge = pl.kernel(
        _edge_body,
        out_type=jax.ShapeDtypeStruct((NC, VP, S), jnp.float32),
        mesh=mesh,
        compiler_params=pltpu.CompilerParams(
            needs_layout_passes=False, use_tc_tiling_on_sc=False),
        scratch_types=[
            pltpu.VMEM((NCHUNK, CH), jnp.int32),
            pltpu.VMEM((NCHUNK, CH), jnp.int32),
            pltpu.VMEM((EPW,), jnp.float32),
            pltpu.VMEM((2, CH, S * S), jnp.float32),
            pltpu.VMEM((2, CH, S * S), jnp.float32),
            pltpu.VMEM((2, CH, S), jnp.float32),
            pltpu.VMEM((CH, S), jnp.float32),
            pltpu.VMEM((VPS // 4, S), jnp.float32),
            pltpu.VMEM_SHARED((VP, S), jnp.float32),
            pltpu.SemaphoreType.DMA,
            pltpu.SemaphoreType.DMA,
            pltpu.SemaphoreType.DMA,
            pltpu.SemaphoreType.DMA,
            pltpu.SemaphoreType.DMA,
            pltpu.SemaphoreType.DMA,
        ],
    )
    return hist, edge


def kernel(feat_Matrix, X_Node, X_Neis, dg_list, W_xi, b_xi, W_rou, b_rou,
           W_out, b_out):
    w_t = W_xi.reshape(S, S, 2 * LN).transpose(1, 0, 2).reshape(S * S, 2 * LN)
    w1 = w_t[:, :LN].T
    w2 = w_t[:, LN:].T
    bxi = b_xi.reshape(S, S).T.reshape(1, S * S)
    wr = W_rou.T
    brou = b_rou.reshape(1, S)
    wo = W_out.T
    bo = b_out.reshape(1, C)

    xn = X_Node.astype(jnp.int32)
    xs = X_Neis.astype(jnp.int32)
    xn_w = xn.reshape(NW, EPW)
    xn_c = xn.reshape(NW, NCHUNK, CH)
    xs_c = xs.reshape(NW, NCHUNK, CH)
    dg_w = dg_list.astype(jnp.float32).reshape(NW, EPW)

    hist_kernel, edge_kernel = _sc_kernels()
    cnt32 = hist_kernel(xn_w)
    cnt_t = cnt32.T
    pb, qb, h1 = _mm_call(feat_Matrix, w1, w2, bxi, wr, brou, cnt_t)
    hacc = edge_kernel(pb, qb, h1, xn_c, xs_c, dg_w)
    return _out_call(hacc, h1, wo, bo)


# speedup vs baseline: 2.4659x; 1.0969x over previous
import functools

import jax
import jax.numpy as jnp
from jax import lax
from jax.experimental import pallas as pl
from jax.experimental.pallas import tpu as pltpu
from jax.experimental.pallas import tpu_sc as plsc
from jax._src.pallas.mosaic import lowering as _mosaic_lowering

_mosaic_lowering.lowering_rules[pltpu.CoreType.SC_VECTOR_SUBCORE][lax.tanh_p] = (
    _mosaic_lowering.lowering_rules[pltpu.CoreType.TC][lax.tanh_p])

V = 10000
E = 160000
LN = 128
S = 32
C = 40
MU = 0.9

NC = 2
NS = 16
NW = NC * NS
EPW = E // NW
CH = 8
NCHUNK = EPW // CH
VP = 10240
VPS = VP // NS
BV = 2000
LANES = 16
UNROLL = 4

def _hist_body(xn_hbm, out_hbm, idx_v, cnt_v):
    cid = lax.axis_index("c")
    sid = lax.axis_index("s")
    wid = sid * NC + cid
    pltpu.sync_copy(xn_hbm.at[wid], idx_v)

    def zero_body(i, carry):
        cnt_v[pl.ds(i * LANES, LANES)] = jnp.zeros((LANES,), jnp.float32)
        return carry

    lax.fori_loop(0, V // LANES, zero_body, 0)

    ones = jnp.ones((LANES,), jnp.float32)

    def hist_body(k, carry):
        v16 = idx_v[pl.ds(k * LANES, LANES)]
        plsc.addupdate_scatter(cnt_v, [v16], ones)
        return carry

    nfull = EPW // LANES
    lax.fori_loop(0, nfull, hist_body, 0)
    rem = EPW - nfull * LANES
    if rem:
        v16 = idx_v[pl.ds(EPW - LANES, LANES)]
        mask = lax.iota(jnp.int32, LANES) >= (LANES - rem)
        plsc.addupdate_scatter(cnt_v, [v16], ones, mask=mask)
    pltpu.sync_copy(cnt_v, out_hbm.at[wid])


def _mm_body(feat, w1, w2, bxi, wr, brou, cnt_t, p_out, q_out, h1_out):
    f = feat[...]
    p_out[...] = jnp.dot(
        f, w1[...], preferred_element_type=jnp.float32).astype(jnp.bfloat16)
    q_out[...] = (jnp.dot(f, w2[...], preferred_element_type=jnp.float32)
                  + bxi[...]).astype(jnp.bfloat16)
    r = jnp.tanh(jnp.dot(f, wr[...], preferred_element_type=jnp.float32) + brou[...])
    cnt = jnp.sum(cnt_t[...], axis=1, keepdims=True)
    h1_out[...] = cnt * r


_mm_call = pl.pallas_call(
    _mm_body,
    grid=(V // BV,),
    in_specs=[
        pl.BlockSpec((BV, LN), lambda i: (i, 0)),
        pl.BlockSpec((LN, S * S), lambda i: (0, 0)),
        pl.BlockSpec((LN, S * S), lambda i: (0, 0)),
        pl.BlockSpec((1, S * S), lambda i: (0, 0)),
        pl.BlockSpec((LN, S), lambda i: (0, 0)),
        pl.BlockSpec((1, S), lambda i: (0, 0)),
        pl.BlockSpec((BV, NW), lambda i: (i, 0)),
    ],
    out_specs=[
        pl.BlockSpec((BV, S * S), lambda i: (i, 0)),
        pl.BlockSpec((BV, S * S), lambda i: (i, 0)),
        pl.BlockSpec((BV, S), lambda i: (i, 0)),
    ],
    out_shape=[
        jax.ShapeDtypeStruct((V, S * S), jnp.bfloat16),
        jax.ShapeDtypeStruct((V, S * S), jnp.bfloat16),
        jax.ShapeDtypeStruct((V, S), jnp.float32),
    ],
)


def _edge_body(pb_hbm, qb_hbm, h1_hbm, xn_hbm, xs_hbm, dg_hbm, out_hbm,
                 idxn, idxs, dgb, pbuf, qbuf, h1g, ubuf, zbuf, hsh,
                 sp0, sp1, sq0, sq1, sh0, sh1):
    cid = lax.axis_index("c")
    sid = lax.axis_index("s")
    wid = sid * NC + cid
    semp = (sp0, sp1)
    semq = (sq0, sq1)
    semh = (sh0, sh1)

    pltpu.sync_copy(xn_hbm.at[wid], idxn)
    pltpu.sync_copy(xs_hbm.at[wid], idxs)
    pltpu.sync_copy(dg_hbm.at[wid], dgb)

    zero = jnp.zeros((LANES,), jnp.float32)

    def zb(i, carry):
        for t in range(S // LANES):
            zbuf[i, pl.ds(t * LANES, LANES)] = zero
        return carry

    lax.fori_loop(0, VPS // 4, zb, 0)

    def zs(k, carry):
        off = pl.multiple_of(sid * VPS + k * (VPS // 4), 8)
        pltpu.sync_copy(zbuf, hsh.at[pl.ds(off, VPS // 4)])
        return carry

    lax.fori_loop(0, 4, zs, 0)
    plsc.subcore_barrier()

    def issue(cc, b):
        pltpu.async_copy(pb_hbm.at[idxn.at[cc]], pbuf.at[b], semp[b])
        pltpu.async_copy(qb_hbm.at[idxs.at[cc]], qbuf.at[b], semq[b])
        pltpu.async_copy(h1_hbm.at[idxn.at[cc]], h1g.at[b], semh[b])

    def compute(cc, b):
        pltpu.make_async_copy(pb_hbm.at[idxn.at[cc]], pbuf.at[b], semp[b]).wait()
        pltpu.make_async_copy(qb_hbm.at[idxs.at[cc]], qbuf.at[b], semq[b]).wait()
        pltpu.make_async_copy(h1_hbm.at[idxn.at[cc]], h1g.at[b], semh[b]).wait()

        def edge_body(r, carry):
            eidx = jnp.full((LANES,), cc * CH + r, jnp.int32)
            dvec = plsc.load_gather(dgb, [eidx])
            svec = (MU / S) / dvec
            hs0 = h1g[b, r, pl.ds(0, LANES)] * svec
            hs1 = h1g[b, r, pl.ds(LANES, LANES)] * svec
            acc0 = [jnp.zeros((LANES,), jnp.float32) for _ in range(4)]
            acc1 = [jnp.zeros((LANES,), jnp.float32) for _ in range(4)]
            for col in range(S):
                base = col * S
                pb2 = pbuf[b, r, pl.ds(base, 2 * LANES)]
                qb2 = qbuf[b, r, pl.ds(base, 2 * LANES)]
                x0, x1 = plsc.unpack(
                    pb2 + qb2, format=plsc.PackFormat.INTERLEAVED)
                t0 = jnp.tanh(x0)
                t1 = jnp.tanh(x1)
                hv = (hs0 if col < LANES else hs1)[col % LANES]
                k = col % 4
                acc0[k] = acc0[k] + t0 * hv
                acc1[k] = acc1[k] + t1 * hv
            u0 = (acc0[0] + acc0[1]) + (acc0[2] + acc0[3])
            u1 = (acc1[0] + acc1[1]) + (acc1[2] + acc1[3])
            ubuf[r, pl.ds(0, LANES)] = u0
            ubuf[r, pl.ds(LANES, LANES)] = u1
            return carry

        lax.fori_loop(0, CH, edge_body, 0)
        pltpu.sync_copy(ubuf, hsh.at[idxn.at[cc]], add=True)

    issue(0, 0)

    def pair_body(k, carry):
        for b in range(2):
            cc = 2 * k + b
            issue(cc + 1, 1 - b)
            compute(cc, b)
        return carry

    lax.fori_loop(0, (NCHUNK - 1) // 2, pair_body, 0)
    compute(NCHUNK - 1, 0)

    plsc.subcore_barrier()
    doff = pl.multiple_of(sid * VPS, 8)
    pltpu.sync_copy(hsh.at[pl.ds(doff, VPS)],
                    out_hbm.at[cid, pl.ds(doff, VPS)])


def _out_body(hacc, h1, wo, bo, out):
    h2 = hacc[0, :, :S] + hacc[1, :, :S] + h1[:, :S]
    logits = jnp.dot(h2, wo[...], preferred_element_type=jnp.float32) + bo[...]
    m = jnp.max(logits, axis=-1, keepdims=True)
    z = logits - m
    lse = jnp.log(jnp.sum(jnp.exp(z), axis=-1, keepdims=True))
    out[...] = z - lse


_out_call = pl.pallas_call(
    _out_body,
    grid=(V // BV,),
    in_specs=[
        pl.BlockSpec((NC, BV, S), lambda i: (0, i, 0)),
        pl.BlockSpec((BV, S), lambda i: (i, 0)),
        pl.BlockSpec((S, C), lambda i: (0, 0)),
        pl.BlockSpec((1, C), lambda i: (0, 0)),
    ],
    out_specs=pl.BlockSpec((BV, C), lambda i: (i, 0)),
    out_shape=jax.ShapeDtypeStruct((V, C), jnp.float32),
)


@functools.lru_cache(maxsize=1)
def _sc_kernels():
    mesh = plsc.VectorSubcoreMesh(
        core_axis_name="c", subcore_axis_name="s",
        num_cores=NC, num_subcores=NS)
    params = pltpu.CompilerParams(needs_layout_passes=False)
    hist = pl.kernel(
        _hist_body,
        out_type=jax.ShapeDtypeStruct((NW, V), jnp.float32),
        mesh=mesh,
        compiler_params=params,
        scratch_types=[
            pltpu.VMEM((EPW,), jnp.int32),
            pltpu.VMEM((V,), jnp.float32),
        ],
    )
    edge = pl.kernel(
        _edge_body,
        out_type=jax.ShapeDtypeStruct((NC, VP, S), jnp.float32),
        mesh=mesh,
        compiler_params=pltpu.CompilerParams(
            needs_layout_passes=False, use_tc_tiling_on_sc=False),
        scratch_types=[
            pltpu.VMEM((NCHUNK, CH), jnp.int32),
            pltpu.VMEM((NCHUNK, CH), jnp.int32),
            pltpu.VMEM((EPW,), jnp.float32),
            pltpu.VMEM((2, CH, S * S), jnp.bfloat16),
            pltpu.VMEM((2, CH, S * S), jnp.bfloat16),
            pltpu.VMEM((2, CH, S), jnp.float32),
            pltpu.VMEM((CH, S), jnp.float32),
            pltpu.VMEM((VPS // 4, S), jnp.float32),
            pltpu.VMEM_SHARED((VP, S), jnp.float32),
            pltpu.SemaphoreType.DMA,
            pltpu.SemaphoreType.DMA,
            pltpu.SemaphoreType.DMA,
            pltpu.SemaphoreType.DMA,
            pltpu.SemaphoreType.DMA,
            pltpu.SemaphoreType.DMA,
        ],
    )
    return hist, edge


def kernel(feat_Matrix, X_Node, X_Neis, dg_list, W_xi, b_xi, W_rou, b_rou,
           W_out, b_out):
    kidx = jnp.arange(S * S, dtype=jnp.int32)
    jcol = kidx // S
    m = kidx % S
    irow = (m % 2) * LANES + m // 2
    perm = irow * S + jcol
    w_t = W_xi[perm]
    w1 = w_t[:, :LN].T
    w2 = w_t[:, LN:].T
    bxi = b_xi[perm].reshape(1, S * S)
    wr = W_rou.T
    brou = b_rou.reshape(1, S)
    wo = W_out.T
    bo = b_out.reshape(1, C)

    xn = X_Node.astype(jnp.int32)
    xs = X_Neis.astype(jnp.int32)
    xn_w = xn.reshape(NW, EPW)
    xn_c = xn.reshape(NW, NCHUNK, CH)
    xs_c = xs.reshape(NW, NCHUNK, CH)
    dg_w = dg_list.astype(jnp.float32).reshape(NW, EPW)

    hist_kernel, edge_kernel = _sc_kernels()
    cnt32 = hist_kernel(xn_w)
    cnt_t = cnt32.T
    pb, qb, h1 = _mm_call(feat_Matrix, w1, w2, bxi, wr, brou, cnt_t)
    hacc = edge_kernel(pb, qb, h1, xn_c, xs_c, dg_w)
    return _out_call(hacc, h1, wo, bo)


# speedup vs baseline: 2.9023x; 1.1770x over previous
import functools

import jax
import jax.numpy as jnp
from jax import lax
from jax.experimental import pallas as pl
from jax.experimental.pallas import tpu as pltpu
from jax.experimental.pallas import tpu_sc as plsc
from jax._src.pallas.mosaic import lowering as _mosaic_lowering

_mosaic_lowering.lowering_rules[pltpu.CoreType.SC_VECTOR_SUBCORE][lax.tanh_p] = (
    _mosaic_lowering.lowering_rules[pltpu.CoreType.TC][lax.tanh_p])

V = 10000
E = 160000
LN = 128
S = 32
C = 40
MU = 0.9

NC = 2
NS = 16
NW = NC * NS
EPW = E // NW
CH = 40
NCHUNK = EPW // CH
VP = 10240
VPS = VP // NS
BV = 2000
LANES = 16
UNROLL = 4

def _hist_body(xn_hbm, out_hbm, idx_v, cnt_v):
    cid = lax.axis_index("c")
    sid = lax.axis_index("s")
    wid = sid * NC + cid
    pltpu.sync_copy(xn_hbm.at[wid], idx_v)

    def zero_body(i, carry):
        cnt_v[pl.ds(i * LANES, LANES)] = jnp.zeros((LANES,), jnp.float32)
        return carry

    lax.fori_loop(0, V // LANES, zero_body, 0)

    ones = jnp.ones((LANES,), jnp.float32)

    def hist_body(k, carry):
        v16 = idx_v[pl.ds(k * LANES, LANES)]
        plsc.addupdate_scatter(cnt_v, [v16], ones)
        return carry

    nfull = EPW // LANES
    lax.fori_loop(0, nfull, hist_body, 0)
    rem = EPW - nfull * LANES
    if rem:
        v16 = idx_v[pl.ds(EPW - LANES, LANES)]
        mask = lax.iota(jnp.int32, LANES) >= (LANES - rem)
        plsc.addupdate_scatter(cnt_v, [v16], ones, mask=mask)
    pltpu.sync_copy(cnt_v, out_hbm.at[wid])


def _mm_body(feat, w1, w2, bxi, wr, brou, cnt_t, p_out, q_out, h1_out):
    f = feat[...]
    p_out[...] = jnp.dot(
        f, w1[...], preferred_element_type=jnp.float32).astype(jnp.bfloat16)
    q_out[...] = (jnp.dot(f, w2[...], preferred_element_type=jnp.float32)
                  + bxi[...]).astype(jnp.bfloat16)
    r = jnp.tanh(jnp.dot(f, wr[...], preferred_element_type=jnp.float32) + brou[...])
    cnt = jnp.sum(cnt_t[...], axis=1, keepdims=True)
    h1_out[...] = cnt * r


_mm_call = pl.pallas_call(
    _mm_body,
    grid=(V // BV,),
    in_specs=[
        pl.BlockSpec((BV, LN), lambda i: (i, 0)),
        pl.BlockSpec((LN, S * S), lambda i: (0, 0)),
        pl.BlockSpec((LN, S * S), lambda i: (0, 0)),
        pl.BlockSpec((1, S * S), lambda i: (0, 0)),
        pl.BlockSpec((LN, S), lambda i: (0, 0)),
        pl.BlockSpec((1, S), lambda i: (0, 0)),
        pl.BlockSpec((BV, NW), lambda i: (i, 0)),
    ],
    out_specs=[
        pl.BlockSpec((BV, S * S), lambda i: (i, 0)),
        pl.BlockSpec((BV, S * S), lambda i: (i, 0)),
        pl.BlockSpec((BV, S), lambda i: (i, 0)),
    ],
    out_shape=[
        jax.ShapeDtypeStruct((V, S * S), jnp.bfloat16),
        jax.ShapeDtypeStruct((V, S * S), jnp.bfloat16),
        jax.ShapeDtypeStruct((V, S), jnp.float32),
    ],
)


def _edge_body(pb_hbm, qb_hbm, h1_hbm, xn_hbm, xs_hbm, dg_hbm, out_hbm,
                 idxn, idxs, dgb, pbuf, qbuf, h1g, ubuf, zbuf, hsh,
                 sp0, sp1, sq0, sq1, sh0, sh1):
    cid = lax.axis_index("c")
    sid = lax.axis_index("s")
    wid = sid * NC + cid
    semp = (sp0, sp1)
    semq = (sq0, sq1)
    semh = (sh0, sh1)

    pltpu.sync_copy(xn_hbm.at[wid], idxn)
    pltpu.sync_copy(xs_hbm.at[wid], idxs)
    pltpu.sync_copy(dg_hbm.at[wid], dgb)

    zero = jnp.zeros((LANES,), jnp.float32)

    def zb(i, carry):
        for t in range(S // LANES):
            zbuf[i, pl.ds(t * LANES, LANES)] = zero
        return carry

    lax.fori_loop(0, VPS // 4, zb, 0)

    def zs(k, carry):
        off = pl.multiple_of(sid * VPS + k * (VPS // 4), 8)
        pltpu.sync_copy(zbuf, hsh.at[pl.ds(off, VPS // 4)])
        return carry

    lax.fori_loop(0, 4, zs, 0)
    plsc.subcore_barrier()

    def issue(cc, b):
        pltpu.async_copy(pb_hbm.at[idxn.at[cc]], pbuf.at[b], semp[b])
        pltpu.async_copy(qb_hbm.at[idxs.at[cc]], qbuf.at[b], semq[b])
        pltpu.async_copy(h1_hbm.at[idxn.at[cc]], h1g.at[b], semh[b])

    def compute(cc, b):
        pltpu.make_async_copy(pb_hbm.at[idxn.at[cc]], pbuf.at[b], semp[b]).wait()
        pltpu.make_async_copy(qb_hbm.at[idxs.at[cc]], qbuf.at[b], semq[b]).wait()
        pltpu.make_async_copy(h1_hbm.at[idxn.at[cc]], h1g.at[b], semh[b]).wait()

        def edge_body(r, carry):
            eidx = jnp.full((LANES,), cc * CH + r, jnp.int32)
            dvec = plsc.load_gather(dgb, [eidx])
            svec = (MU / S) / dvec
            hs0 = h1g[b, r, pl.ds(0, LANES)] * svec
            hs1 = h1g[b, r, pl.ds(LANES, LANES)] * svec
            acc0 = [jnp.zeros((LANES,), jnp.float32) for _ in range(4)]
            acc1 = [jnp.zeros((LANES,), jnp.float32) for _ in range(4)]
            for col in range(S):
                base = col * S
                pb2 = pbuf[b, r, pl.ds(base, 2 * LANES)]
                qb2 = qbuf[b, r, pl.ds(base, 2 * LANES)]
                x0, x1 = plsc.unpack(
                    pb2 + qb2, format=plsc.PackFormat.INTERLEAVED)
                t0 = jnp.tanh(x0)
                t1 = jnp.tanh(x1)
                hv = (hs0 if col < LANES else hs1)[col % LANES]
                k = col % 4
                acc0[k] = acc0[k] + t0 * hv
                acc1[k] = acc1[k] + t1 * hv
            u0 = (acc0[0] + acc0[1]) + (acc0[2] + acc0[3])
            u1 = (acc1[0] + acc1[1]) + (acc1[2] + acc1[3])
            ubuf[r, pl.ds(0, LANES)] = u0
            ubuf[r, pl.ds(LANES, LANES)] = u1
            return carry

        lax.fori_loop(0, CH, edge_body, 0)
        pltpu.sync_copy(ubuf, hsh.at[idxn.at[cc]], add=True)

    issue(0, 0)

    def pair_body(k, carry):
        for b in range(2):
            cc = 2 * k + b
            issue(cc + 1, 1 - b)
            compute(cc, b)
        return carry

    lax.fori_loop(0, (NCHUNK - 1) // 2, pair_body, 0)
    compute(NCHUNK - 1, 0)

    plsc.subcore_barrier()
    doff = pl.multiple_of(sid * VPS, 8)
    pltpu.sync_copy(hsh.at[pl.ds(doff, VPS)],
                    out_hbm.at[cid, pl.ds(doff, VPS)])


def _out_body(hacc, h1, wo, bo, out):
    h2 = hacc[0, :, :S] + hacc[1, :, :S] + h1[:, :S]
    logits = jnp.dot(h2, wo[...], preferred_element_type=jnp.float32) + bo[...]
    m = jnp.max(logits, axis=-1, keepdims=True)
    z = logits - m
    lse = jnp.log(jnp.sum(jnp.exp(z), axis=-1, keepdims=True))
    out[...] = z - lse


_out_call = pl.pallas_call(
    _out_body,
    grid=(V // BV,),
    in_specs=[
        pl.BlockSpec((NC, BV, S), lambda i: (0, i, 0)),
        pl.BlockSpec((BV, S), lambda i: (i, 0)),
        pl.BlockSpec((S, C), lambda i: (0, 0)),
        pl.BlockSpec((1, C), lambda i: (0, 0)),
    ],
    out_specs=pl.BlockSpec((BV, C), lambda i: (i, 0)),
    out_shape=jax.ShapeDtypeStruct((V, C), jnp.float32),
)


@functools.lru_cache(maxsize=1)
def _sc_kernels():
    mesh = plsc.VectorSubcoreMesh(
        core_axis_name="c", subcore_axis_name="s",
        num_cores=NC, num_subcores=NS)
    params = pltpu.CompilerParams(needs_layout_passes=False)
    hist = pl.kernel(
        _hist_body,
        out_type=jax.ShapeDtypeStruct((NW, V), jnp.float32),
        mesh=mesh,
        compiler_params=params,
        scratch_types=[
            pltpu.VMEM((EPW,), jnp.int32),
            pltpu.VMEM((V,), jnp.float32),
        ],
    )
    edge = pl.kernel(
        _edge_body,
        out_type=jax.ShapeDtypeStruct((NC, VP, S), jnp.float32),
        mesh=mesh,
        compiler_params=pltpu.CompilerParams(
            needs_layout_passes=False, use_tc_tiling_on_sc=False),
        scratch_types=[
            pltpu.VMEM((NCHUNK, CH), jnp.int32),
            pltpu.VMEM((NCHUNK, CH), jnp.int32),
            pltpu.VMEM((EPW,), jnp.float32),
            pltpu.VMEM((2, CH, S * S), jnp.bfloat16),
            pltpu.VMEM((2, CH, S * S), jnp.bfloat16),
            pltpu.VMEM((2, CH, S), jnp.float32),
            pltpu.VMEM((CH, S), jnp.float32),
            pltpu.VMEM((VPS // 4, S), jnp.float32),
            pltpu.VMEM_SHARED((VP, S), jnp.float32),
            pltpu.SemaphoreType.DMA,
            pltpu.SemaphoreType.DMA,
            pltpu.SemaphoreType.DMA,
            pltpu.SemaphoreType.DMA,
            pltpu.SemaphoreType.DMA,
            pltpu.SemaphoreType.DMA,
        ],
    )
    return hist, edge


def kernel(feat_Matrix, X_Node, X_Neis, dg_list, W_xi, b_xi, W_rou, b_rou,
           W_out, b_out):
    kidx = jnp.arange(S * S, dtype=jnp.int32)
    jcol = kidx // S
    m = kidx % S
    irow = (m % 2) * LANES + m // 2
    perm = irow * S + jcol
    w_t = W_xi[perm]
    w1 = w_t[:, :LN].T
    w2 = w_t[:, LN:].T
    bxi = b_xi[perm].reshape(1, S * S)
    wr = W_rou.T
    brou = b_rou.reshape(1, S)
    wo = W_out.T
    bo = b_out.reshape(1, C)

    xn = X_Node.astype(jnp.int32)
    xs = X_Neis.astype(jnp.int32)
    xn_w = xn.reshape(NW, EPW)
    xn_c = xn.reshape(NW, NCHUNK, CH)
    xs_c = xs.reshape(NW, NCHUNK, CH)
    dg_w = dg_list.astype(jnp.float32).reshape(NW, EPW)

    hist_kernel, edge_kernel = _sc_kernels()
    cnt32 = hist_kernel(xn_w)
    cnt_t = cnt32.T
    pb, qb, h1 = _mm_call(feat_Matrix, w1, w2, bxi, wr, brou, cnt_t)
    hacc = edge_kernel(pb, qb, h1, xn_c, xs_c, dg_w)
    return _out_call(hacc, h1, wo, bo)
